# baseline (speedup 1.0000x reference)
import functools

import jax
import jax.numpy as jnp
from jax import lax
from jax.experimental import pallas as pl
from jax.experimental.pallas import tpu as pltpu
from jax.experimental.pallas import tpu_sc as plsc

IN_F = 1024
OUT_F = 4096
BLK = 256
NBLK = OUT_F // BLK
TOPK = NBLK // 4
TOK = 2048
R = 128
G = (TOK * TOPK + NBLK * (R - 1) + R - 1) // R
GPAD = 128
P = G * R
NW = 32
TPW = TOK // NW
NSUB = TPW // 16


def _router_body(x_ref, wr_ref, br_ref, dest_ref, bid_ref):
    xt = x_ref[...]
    logits = lax.dot_general(xt, wr_ref[...], (((1,), (1,)), ((), ())),
                             preferred_element_type=jnp.float32)
    logits = logits + br_ref[...]
    cols = lax.broadcasted_iota(jnp.int32, (TOK, NBLK), 1)
    work = logits
    sels = []
    for _ in range(TOPK):
        m = jnp.max(work, axis=1, keepdims=True)
        idxk = jnp.min(jnp.where(work == m, cols, NBLK), axis=1, keepdims=True)
        sel = cols == idxk
        sels.append(sel)
        work = jnp.where(sel, -jnp.inf, work)
    maskf = sels[0].astype(jnp.float32)
    for sel in sels[1:]:
        maskf = maskf + sel.astype(jnp.float32)

    counts = jnp.sum(maskf, axis=0, keepdims=True)
    ntiles = jnp.floor((counts + (R - 1)) * (1.0 / R))
    padded = ntiles * R

    tri_u = (lax.broadcasted_iota(jnp.int32, (NBLK, NBLK), 0)
             <= lax.broadcasted_iota(jnp.int32, (NBLK, NBLK), 1)
             ).astype(jnp.float32)

    def _lanecumsum(v):
        return lax.dot_general(v, tri_u, (((1,), (0,)), ((), ())),
                               preferred_element_type=jnp.float32,
                               precision=lax.Precision.HIGHEST)

    offs_incl = _lanecumsum(padded)
    offs_excl = offs_incl - padded

    CH = 256
    tri_l = (lax.broadcasted_iota(jnp.int32, (CH, CH), 0)
             >= lax.broadcasted_iota(jnp.int32, (CH, CH), 1)
             ).astype(jnp.float32)
    pieces = []
    carry = jnp.zeros((1, NBLK), jnp.float32)
    for c in range(TOK // CH):
        chunk = maskf[c * CH:(c + 1) * CH, :]
        local = lax.dot_general(tri_l, chunk, (((1,), (0,)), ((), ())),
                                preferred_element_type=jnp.float32,
                                precision=lax.Precision.HIGHEST)
        pieces.append(local + carry)
        carry = carry + local[CH - 1:CH, :]
    csum_incl = jnp.concatenate(pieces, axis=0)
    csum_excl = csum_incl - maskf
    destmat = offs_excl + csum_excl
    dcols = [jnp.sum(jnp.where(sel, destmat, 0.0), axis=1, keepdims=True)
             for sel in sels]
    dest4 = jnp.concatenate(dcols, axis=1)
    TC = 256
    eye_t = (lax.broadcasted_iota(jnp.int32, (TC, TC), 0)
             == lax.broadcasted_iota(jnp.int32, (TC, TC), 1)
             ).astype(jnp.float32)
    tchunks = []
    for c in range(TOK // TC):
        a = dest4[c * TC:(c + 1) * TC, :]
        tchunks.append(lax.dot_general(
            a, eye_t, (((0,), (0,)), ((), ())),
            preferred_element_type=jnp.float32,
            precision=lax.Precision.HIGHEST))
    dest_ref[...] = jnp.concatenate(tchunks, axis=1).astype(jnp.int32)

    cumt = _lanecumsum(ntiles)
    eye = (lax.broadcasted_iota(jnp.int32, (NBLK, NBLK), 0)
           == lax.broadcasted_iota(jnp.int32, (NBLK, NBLK), 1)
           ).astype(jnp.float32)
    cumt_col = lax.dot_general(eye, cumt, (((1,), (1,)), ((), ())),
                               preferred_element_type=jnp.float32,
                               precision=lax.Precision.HIGHEST)
    j_row = lax.broadcasted_iota(jnp.int32, (1, GPAD), 1)
    cmp = (cumt_col.astype(jnp.int32) <= j_row).astype(jnp.int32)
    bid = jnp.sum(cmp, axis=0, keepdims=True)
    bid_ref[...] = jnp.minimum(bid, NBLK - 1)


def _router(x2, Wr, br2):
    return pl.pallas_call(
        _router_body,
        out_shape=(jax.ShapeDtypeStruct((TOPK, TOK), jnp.int32),
                   jax.ShapeDtypeStruct((1, GPAD), jnp.int32)),
    )(x2, Wr, br2)


def _dispatch_body(x_hbm, destT_hbm, xg_hbm, rows_v, idx_v, sem):
    wid = lax.axis_index("c") * 16 + lax.axis_index("s")
    tok0 = wid * TPW
    pltpu.sync_copy(x_hbm.at[pl.ds(tok0, TPW), :], rows_v)
    for k in range(TOPK):
        pltpu.sync_copy(destT_hbm.at[k, pl.ds(tok0, TPW)], idx_v.at[k])
    cps = [pltpu.async_copy(rows_v, xg_hbm.at[idx_v.at[k]], sem)
           for k in range(TOPK)]
    for cp in cps:
        cp.wait()


def _dispatch(x2, dest):
    mesh = plsc.VectorSubcoreMesh(core_axis_name="c", subcore_axis_name="s")
    fn = functools.partial(
        pl.kernel,
        out_type=jax.ShapeDtypeStruct((P, IN_F), jnp.float32),
        mesh=mesh,
        scratch_types=[
            pltpu.VMEM((TPW, IN_F), jnp.float32),
            pltpu.VMEM((TOPK, TPW), jnp.int32),
            pltpu.SemaphoreType.DMA,
        ],
    )(_dispatch_body)
    return fn(x2, dest)


def _gmm_body(bid_ref, xg_ref, w1_ref, b1_ref, w2_ref, b2_ref, yg_ref):
    xt = xg_ref[...]
    h = lax.dot_general(xt, w1_ref[...], (((1,), (1,)), ((), ())),
                        preferred_element_type=jnp.float32)
    h = jnp.maximum(h + b1_ref[0], 0.0)
    yg = lax.dot_general(h, w2_ref[...], (((1,), (1,)), ((), ())),
                         preferred_element_type=jnp.float32)
    yg_ref[...] = yg + 0.25 * b2_ref[...]


def _grouped_ffn(bid, xg, W1, b1b, W2, b22):
    grid_spec = pltpu.PrefetchScalarGridSpec(
        num_scalar_prefetch=1,
        grid=(G,),
        in_specs=[
            pl.BlockSpec((R, IN_F), lambda j, b: (j, 0)),
            pl.BlockSpec((BLK, IN_F), lambda j, b: (b[0, j], 0)),
            pl.BlockSpec((1, 1, BLK), lambda j, b: (b[0, j], 0, 0)),
            pl.BlockSpec((IN_F, BLK), lambda j, b: (0, b[0, j])),
            pl.BlockSpec((1, IN_F), lambda j, b: (0, 0)),
        ],
        out_specs=pl.BlockSpec((R, IN_F), lambda j, b: (j, 0)),
    )
    return pl.pallas_call(
        _gmm_body,
        grid_spec=grid_spec,
        out_shape=jax.ShapeDtypeStruct((P, IN_F), jnp.float32),
    )(bid, xg, W1, b1b, W2, b22)


def _combine_body(yg_hbm, destT_hbm, y_hbm, idx_v,
                  buf0, buf1, buf2, buf3, acc, sem):
    wid = lax.axis_index("c") * 16 + lax.axis_index("s")
    tok0 = wid * TPW
    for k in range(TOPK):
        for c4 in range(NSUB):
            pltpu.sync_copy(destT_hbm.at[k, pl.ds(tok0 + c4 * 16, 16)],
                            idx_v.at[k * NSUB + c4])
    bufs = (buf0, buf1, buf2, buf3)
    for sub in range(NSUB):
        cps = [pltpu.async_copy(yg_hbm.at[idx_v.at[k * NSUB + sub]],
                                bufs[k], sem) for k in range(TOPK)]
        for cp in cps:
            cp.wait()
        for r in range(16):
            def cc_body(cc, _, _r=r):
                sl = pl.ds(cc * 16, 16)
                acc[_r, sl] = (buf0[_r, sl] + buf1[_r, sl]
                               + buf2[_r, sl] + buf3[_r, sl])
                return 0
            lax.fori_loop(0, IN_F // 16, cc_body, 0, unroll=4)
        pltpu.sync_copy(acc, y_hbm.at[pl.ds(tok0 + sub * 16, 16), :])


def _combine(yg, dest):
    mesh = plsc.VectorSubcoreMesh(core_axis_name="c", subcore_axis_name="s")
    fn = functools.partial(
        pl.kernel,
        out_type=jax.ShapeDtypeStruct((TOK, IN_F), jnp.float32),
        mesh=mesh,
        scratch_types=[
            pltpu.VMEM((TOPK * NSUB, 16), jnp.int32),
            pltpu.VMEM((16, IN_F), jnp.float32),
            pltpu.VMEM((16, IN_F), jnp.float32),
            pltpu.VMEM((16, IN_F), jnp.float32),
            pltpu.VMEM((16, IN_F), jnp.float32),
            pltpu.VMEM((16, IN_F), jnp.float32),
            pltpu.SemaphoreType.DMA,
        ],
    )(_combine_body)
    return fn(yg, dest)


def kernel(x, Wr, br, W1, b1, W2, b2):
    x2 = x.reshape(TOK, IN_F)
    br2 = br.reshape(1, NBLK)
    b1b = b1.reshape(NBLK, 1, BLK)
    b22 = b2.reshape(1, IN_F)

    dest, bid = _router(x2, Wr, br2)
    xg = _dispatch(x2, dest)
    yg = _grouped_ffn(bid, xg, W1, b1b, W2, b22)
    y = _combine(yg, dest)
    return y.reshape(x.shape)


# speedup vs baseline: 1.0925x; 1.0925x over previous
import functools

import jax
import jax.numpy as jnp
from jax import lax
from jax.experimental import pallas as pl
from jax.experimental.pallas import tpu as pltpu
from jax.experimental.pallas import tpu_sc as plsc

IN_F = 1024
OUT_F = 4096
BLK = 256
NBLK = OUT_F // BLK
TOPK = NBLK // 4
TOK = 2048
R = 128
G = (TOK * TOPK + NBLK * (R - 1) + R - 1) // R
GPAD = 128
P = G * R
NW = 32
TPW = TOK // NW
NSUB = TPW // 16


def _router_body(x_ref, wr_ref, br_ref, dest_ref, bid_ref):
    xt = x_ref[...]
    logits = lax.dot_general(xt, wr_ref[...], (((1,), (1,)), ((), ())),
                             preferred_element_type=jnp.float32)
    logits = logits + br_ref[...]
    cols = lax.broadcasted_iota(jnp.int32, (TOK, NBLK), 1)
    work = logits
    sels = []
    for _ in range(TOPK):
        m = jnp.max(work, axis=1, keepdims=True)
        idxk = jnp.min(jnp.where(work == m, cols, NBLK), axis=1, keepdims=True)
        sel = cols == idxk
        sels.append(sel)
        work = jnp.where(sel, -jnp.inf, work)
    maskf = sels[0].astype(jnp.float32)
    for sel in sels[1:]:
        maskf = maskf + sel.astype(jnp.float32)

    counts = jnp.sum(maskf, axis=0, keepdims=True)
    ntiles = jnp.floor((counts + (R - 1)) * (1.0 / R))
    padded = ntiles * R

    tri_u = (lax.broadcasted_iota(jnp.int32, (NBLK, NBLK), 0)
             <= lax.broadcasted_iota(jnp.int32, (NBLK, NBLK), 1)
             ).astype(jnp.float32)

    def _lanecumsum(v):
        return lax.dot_general(v, tri_u, (((1,), (0,)), ((), ())),
                               preferred_element_type=jnp.float32,
                               precision=lax.Precision.HIGHEST)

    offs_incl = _lanecumsum(padded)
    offs_excl = offs_incl - padded

    CH = 256
    tri_l = (lax.broadcasted_iota(jnp.int32, (CH, CH), 0)
             >= lax.broadcasted_iota(jnp.int32, (CH, CH), 1)
             ).astype(jnp.float32)
    pieces = []
    carry = jnp.zeros((1, NBLK), jnp.float32)
    for c in range(TOK // CH):
        chunk = maskf[c * CH:(c + 1) * CH, :]
        local = lax.dot_general(tri_l, chunk, (((1,), (0,)), ((), ())),
                                preferred_element_type=jnp.float32,
                                precision=lax.Precision.HIGHEST)
        pieces.append(local + carry)
        carry = carry + local[CH - 1:CH, :]
    csum_incl = jnp.concatenate(pieces, axis=0)
    csum_excl = csum_incl - maskf
    destmat = offs_excl + csum_excl
    dcols = [jnp.sum(jnp.where(sel, destmat, 0.0), axis=1, keepdims=True)
             for sel in sels]
    dest4 = jnp.concatenate(dcols, axis=1)
    TC = 256
    eye_t = (lax.broadcasted_iota(jnp.int32, (TC, TC), 0)
             == lax.broadcasted_iota(jnp.int32, (TC, TC), 1)
             ).astype(jnp.float32)
    tchunks = []
    for c in range(TOK // TC):
        a = dest4[c * TC:(c + 1) * TC, :]
        tchunks.append(lax.dot_general(
            a, eye_t, (((0,), (0,)), ((), ())),
            preferred_element_type=jnp.float32,
            precision=lax.Precision.HIGHEST))
    dest_ref[...] = jnp.concatenate(tchunks, axis=1).astype(jnp.int32)

    cumt = _lanecumsum(ntiles)
    eye = (lax.broadcasted_iota(jnp.int32, (NBLK, NBLK), 0)
           == lax.broadcasted_iota(jnp.int32, (NBLK, NBLK), 1)
           ).astype(jnp.float32)
    cumt_col = lax.dot_general(eye, cumt, (((1,), (1,)), ((), ())),
                               preferred_element_type=jnp.float32,
                               precision=lax.Precision.HIGHEST)
    j_row = lax.broadcasted_iota(jnp.int32, (1, GPAD), 1)
    cmp = (cumt_col.astype(jnp.int32) <= j_row).astype(jnp.int32)
    bid = jnp.sum(cmp, axis=0, keepdims=True)
    bid_ref[...] = jnp.minimum(bid, NBLK - 1)


def _router(x2, Wr, br2):
    return pl.pallas_call(
        _router_body,
        out_shape=(jax.ShapeDtypeStruct((TOPK, TOK), jnp.int32),
                   jax.ShapeDtypeStruct((1, GPAD), jnp.int32)),
    )(x2, Wr, br2)


def _dispatch_body(x_hbm, destT_hbm, xg_hbm, rows_v, idx_v, sem):
    wid = lax.axis_index("c") * 16 + lax.axis_index("s")
    tok0 = wid * TPW
    pltpu.sync_copy(x_hbm.at[pl.ds(tok0, TPW), :], rows_v)
    for k in range(TOPK):
        pltpu.sync_copy(destT_hbm.at[k, pl.ds(tok0, TPW)], idx_v.at[k])
    cps = [pltpu.async_copy(rows_v, xg_hbm.at[idx_v.at[k]], sem)
           for k in range(TOPK)]
    for cp in cps:
        cp.wait()


def _dispatch(x2, dest):
    mesh = plsc.VectorSubcoreMesh(core_axis_name="c", subcore_axis_name="s")
    fn = functools.partial(
        pl.kernel,
        out_type=jax.ShapeDtypeStruct((P, IN_F), jnp.float32),
        mesh=mesh,
        scratch_types=[
            pltpu.VMEM((TPW, IN_F), jnp.float32),
            pltpu.VMEM((TOPK, TPW), jnp.int32),
            pltpu.SemaphoreType.DMA,
        ],
    )(_dispatch_body)
    return fn(x2, dest)


def _gmm_body(bid_ref, xg_ref, w1_ref, b1_ref, w2_ref, b2_ref, yg_ref):
    xt = xg_ref[...]
    h = lax.dot_general(xt, w1_ref[...], (((1,), (1,)), ((), ())),
                        preferred_element_type=jnp.float32)
    h = jnp.maximum(h + b1_ref[0], 0.0)
    yg = lax.dot_general(h, w2_ref[...], (((1,), (1,)), ((), ())),
                         preferred_element_type=jnp.float32)
    yg_ref[...] = yg + 0.25 * b2_ref[...]


def _grouped_ffn(bid, xg, W1, b1b, W2, b22):
    grid_spec = pltpu.PrefetchScalarGridSpec(
        num_scalar_prefetch=1,
        grid=(G,),
        in_specs=[
            pl.BlockSpec((R, IN_F), lambda j, b: (j, 0)),
            pl.BlockSpec((BLK, IN_F), lambda j, b: (b[0, j], 0)),
            pl.BlockSpec((1, 1, BLK), lambda j, b: (b[0, j], 0, 0)),
            pl.BlockSpec((IN_F, BLK), lambda j, b: (0, b[0, j])),
            pl.BlockSpec((1, IN_F), lambda j, b: (0, 0)),
        ],
        out_specs=pl.BlockSpec((R, IN_F), lambda j, b: (j, 0)),
    )
    return pl.pallas_call(
        _gmm_body,
        grid_spec=grid_spec,
        out_shape=jax.ShapeDtypeStruct((P, IN_F), jnp.float32),
    )(bid, xg, W1, b1b, W2, b22)


CSUB = 8
CNS = TPW // CSUB


def _combine_body(yg_hbm, destT_hbm, y_hbm, idx_v,
                  a0, a1, a2, a3, b0, b1, b2, b3,
                  gsem_a, gsem_b, wsem_a, wsem_b):
    wid = lax.axis_index("c") * 16 + lax.axis_index("s")
    tok0 = wid * TPW
    for k in range(TOPK):
        for c in range(CNS):
            pltpu.sync_copy(destT_hbm.at[k, pl.ds(tok0 + c * CSUB, CSUB)],
                            idx_v.at[k * CNS + c])
    sets = ((a0, a1, a2, a3, gsem_a, wsem_a),
            (b0, b1, b2, b3, gsem_b, wsem_b))

    def fire(sub, s):
        bufs = sets[s]
        for k in range(TOPK):
            pltpu.async_copy(yg_hbm.at[idx_v.at[k * CNS + sub]],
                             bufs[k], bufs[4])

    fire(0, 0)
    for sub in range(CNS):
        s = sub & 1
        bufs = sets[s]
        other = sets[1 - s]
        for k in range(TOPK):
            pltpu.make_async_copy(yg_hbm.at[idx_v.at[k * CNS + sub]],
                                  bufs[k], bufs[4]).wait()
        if sub + 1 < CNS:
            if sub >= 1:
                pltpu.make_async_copy(
                    other[0],
                    y_hbm.at[pl.ds(tok0 + (sub - 1) * CSUB, CSUB), :],
                    other[5]).wait()
            fire(sub + 1, 1 - s)

        @plsc.parallel_loop(0, CSUB * (IN_F // 16), unroll=8)
        def _add(i):
            r = i // (IN_F // 16)
            sl = pl.ds((i % (IN_F // 16)) * 16, 16)
            bufs[0][r, sl] = (bufs[0][r, sl] + bufs[1][r, sl]
                              + bufs[2][r, sl] + bufs[3][r, sl])

        pltpu.async_copy(bufs[0],
                         y_hbm.at[pl.ds(tok0 + sub * CSUB, CSUB), :],
                         bufs[5])
    for sub in (CNS - 2, CNS - 1):
        bufs = sets[sub & 1]
        pltpu.make_async_copy(bufs[0],
                              y_hbm.at[pl.ds(tok0 + sub * CSUB, CSUB), :],
                              bufs[5]).wait()


def _combine(yg, dest):
    mesh = plsc.VectorSubcoreMesh(core_axis_name="c", subcore_axis_name="s")
    fn = functools.partial(
        pl.kernel,
        out_type=jax.ShapeDtypeStruct((TOK, IN_F), jnp.float32),
        mesh=mesh,
        scratch_types=(
            [pltpu.VMEM((TOPK * CNS, CSUB), jnp.int32)]
            + [pltpu.VMEM((CSUB, IN_F), jnp.float32) for _ in range(8)]
            + [pltpu.SemaphoreType.DMA for _ in range(4)]
        ),
    )(_combine_body)
    return fn(yg, dest)


def kernel(x, Wr, br, W1, b1, W2, b2):
    x2 = x.reshape(TOK, IN_F)
    br2 = br.reshape(1, NBLK)
    b1b = b1.reshape(NBLK, 1, BLK)
    b22 = b2.reshape(1, IN_F)

    dest, bid = _router(x2, Wr, br2)
    xg = _dispatch(x2, dest)
    yg = _grouped_ffn(bid, xg, W1, b1b, W2, b22)
    y = _combine(yg, dest)
    return y.reshape(x.shape)


# speedup vs baseline: 1.2958x; 1.1861x over previous
import functools

import jax
import jax.numpy as jnp
from jax import lax
from jax.experimental import pallas as pl
from jax.experimental.pallas import tpu as pltpu
from jax.experimental.pallas import tpu_sc as plsc

IN_F = 1024
OUT_F = 4096
BLK = 256
NBLK = OUT_F // BLK
TOPK = NBLK // 4
TOK = 2048
R = 256
G = (TOK * TOPK + NBLK * (R - 1) + R - 1) // R
GPAD = 128
P = G * R
NW = 32
TPW = TOK // NW
NSUB = TPW // 16


def _router_body(x_ref, wr_ref, br_ref, dest_ref, bid_ref):
    xt = x_ref[...]
    logits = lax.dot_general(xt, wr_ref[...], (((1,), (1,)), ((), ())),
                             preferred_element_type=jnp.float32)
    logits = logits + br_ref[...]
    cols = lax.broadcasted_iota(jnp.int32, (TOK, NBLK), 1)
    work = logits
    sels = []
    for _ in range(TOPK):
        m = jnp.max(work, axis=1, keepdims=True)
        idxk = jnp.min(jnp.where(work == m, cols, NBLK), axis=1, keepdims=True)
        sel = cols == idxk
        sels.append(sel)
        work = jnp.where(sel, -jnp.inf, work)
    maskf = sels[0].astype(jnp.float32)
    for sel in sels[1:]:
        maskf = maskf + sel.astype(jnp.float32)

    counts = jnp.sum(maskf, axis=0, keepdims=True)
    ntiles = jnp.floor((counts + (R - 1)) * (1.0 / R))
    padded = ntiles * R

    tri_u = (lax.broadcasted_iota(jnp.int32, (NBLK, NBLK), 0)
             <= lax.broadcasted_iota(jnp.int32, (NBLK, NBLK), 1)
             ).astype(jnp.float32)

    def _lanecumsum(v):
        return lax.dot_general(v, tri_u, (((1,), (0,)), ((), ())),
                               preferred_element_type=jnp.float32,
                               precision=lax.Precision.HIGHEST)

    offs_incl = _lanecumsum(padded)
    offs_excl = offs_incl - padded

    CH = 256
    tri_l = (lax.broadcasted_iota(jnp.int32, (CH, CH), 0)
             >= lax.broadcasted_iota(jnp.int32, (CH, CH), 1)
             ).astype(jnp.bfloat16)
    pieces = []
    carry = jnp.zeros((1, NBLK), jnp.float32)
    for c in range(TOK // CH):
        chunk = maskf[c * CH:(c + 1) * CH, :].astype(jnp.bfloat16)
        local = lax.dot_general(tri_l, chunk, (((1,), (0,)), ((), ())),
                                preferred_element_type=jnp.float32)
        pieces.append(local + carry)
        carry = carry + local[CH - 1:CH, :]
    csum_incl = jnp.concatenate(pieces, axis=0)
    csum_excl = csum_incl - maskf
    destmat = offs_excl + csum_excl
    dcols = [jnp.sum(jnp.where(sel, destmat, 0.0), axis=1, keepdims=True)
             for sel in sels]
    dest4 = jnp.concatenate(dcols, axis=1)
    TC = 256
    eye_t = (lax.broadcasted_iota(jnp.int32, (TC, TC), 0)
             == lax.broadcasted_iota(jnp.int32, (TC, TC), 1)
             ).astype(jnp.float32)
    tchunks = []
    for c in range(TOK // TC):
        a = dest4[c * TC:(c + 1) * TC, :]
        tchunks.append(lax.dot_general(
            a, eye_t, (((0,), (0,)), ((), ())),
            preferred_element_type=jnp.float32,
            precision=lax.Precision.HIGHEST))
    dest_ref[...] = jnp.concatenate(tchunks, axis=1).astype(jnp.int32)

    cumt = _lanecumsum(ntiles)
    eye = (lax.broadcasted_iota(jnp.int32, (NBLK, NBLK), 0)
           == lax.broadcasted_iota(jnp.int32, (NBLK, NBLK), 1)
           ).astype(jnp.float32)
    cumt_col = lax.dot_general(eye, cumt, (((1,), (1,)), ((), ())),
                               preferred_element_type=jnp.float32,
                               precision=lax.Precision.HIGHEST)
    j_row = lax.broadcasted_iota(jnp.int32, (1, GPAD), 1)
    cmp = (cumt_col.astype(jnp.int32) <= j_row).astype(jnp.int32)
    bid = jnp.sum(cmp, axis=0, keepdims=True)
    bid_ref[...] = jnp.minimum(bid, NBLK - 1)


def _router(x2, Wr, br2):
    return pl.pallas_call(
        _router_body,
        out_shape=(jax.ShapeDtypeStruct((TOPK, TOK), jnp.int32),
                   jax.ShapeDtypeStruct((1, GPAD), jnp.int32)),
    )(x2, Wr, br2)


def _dispatch_body(x_hbm, destT_hbm, xg_hbm, rows_v, idx_v, sem):
    wid = lax.axis_index("c") * 16 + lax.axis_index("s")
    tok0 = wid * TPW
    pltpu.sync_copy(x_hbm.at[pl.ds(tok0, TPW), :], rows_v)
    for k in range(TOPK):
        pltpu.sync_copy(destT_hbm.at[k, pl.ds(tok0, TPW)], idx_v.at[k])
    cps = [pltpu.async_copy(rows_v, xg_hbm.at[idx_v.at[k]], sem)
           for k in range(TOPK)]
    for cp in cps:
        cp.wait()


def _dispatch(x2, dest):
    mesh = plsc.VectorSubcoreMesh(core_axis_name="c", subcore_axis_name="s")
    fn = functools.partial(
        pl.kernel,
        out_type=jax.ShapeDtypeStruct((P, IN_F), jnp.float32),
        mesh=mesh,
        scratch_types=[
            pltpu.VMEM((TPW, IN_F), jnp.float32),
            pltpu.VMEM((TOPK, TPW), jnp.int32),
            pltpu.SemaphoreType.DMA,
        ],
    )(_dispatch_body)
    return fn(x2, dest)


def _gmm_body(bid_ref, xg_ref, w1_ref, b1_ref, w2_ref, b2_ref, yg_ref):
    xt = xg_ref[...]
    h = lax.dot_general(xt, w1_ref[...], (((1,), (1,)), ((), ())),
                        preferred_element_type=jnp.float32)
    h = jnp.maximum(h + b1_ref[0], 0.0)
    yg = lax.dot_general(h, w2_ref[...], (((1,), (1,)), ((), ())),
                         preferred_element_type=jnp.float32)
    yg_ref[...] = yg + 0.25 * b2_ref[...]


def _grouped_ffn(bid, xg, W1, b1b, W2, b22):
    grid_spec = pltpu.PrefetchScalarGridSpec(
        num_scalar_prefetch=1,
        grid=(G,),
        in_specs=[
            pl.BlockSpec((R, IN_F), lambda j, b: (j, 0)),
            pl.BlockSpec((BLK, IN_F), lambda j, b: (b[0, j], 0)),
            pl.BlockSpec((1, 1, BLK), lambda j, b: (b[0, j], 0, 0)),
            pl.BlockSpec((IN_F, BLK), lambda j, b: (0, b[0, j])),
            pl.BlockSpec((1, IN_F), lambda j, b: (0, 0)),
        ],
        out_specs=pl.BlockSpec((R, IN_F), lambda j, b: (j, 0)),
    )
    return pl.pallas_call(
        _gmm_body,
        grid_spec=grid_spec,
        out_shape=jax.ShapeDtypeStruct((P, IN_F), jnp.float32),
    )(bid, xg, W1, b1b, W2, b22)


CSUB = 8
CNS = TPW // CSUB


def _combine_body(yg_hbm, destT_hbm, y_hbm, idx_v,
                  a0, a1, a2, a3, b0, b1, b2, b3,
                  gsem_a, gsem_b, wsem_a, wsem_b):
    wid = lax.axis_index("c") * 16 + lax.axis_index("s")
    tok0 = wid * TPW
    for k in range(TOPK):
        for c in range(CNS):
            pltpu.sync_copy(destT_hbm.at[k, pl.ds(tok0 + c * CSUB, CSUB)],
                            idx_v.at[k * CNS + c])
    sets = ((a0, a1, a2, a3, gsem_a, wsem_a),
            (b0, b1, b2, b3, gsem_b, wsem_b))

    def fire(sub, s):
        bufs = sets[s]
        for k in range(TOPK):
            pltpu.async_copy(yg_hbm.at[idx_v.at[k * CNS + sub]],
                             bufs[k], bufs[4])

    fire(0, 0)
    for sub in range(CNS):
        s = sub & 1
        bufs = sets[s]
        other = sets[1 - s]
        for k in range(TOPK):
            pltpu.make_async_copy(yg_hbm.at[idx_v.at[k * CNS + sub]],
                                  bufs[k], bufs[4]).wait()
        if sub + 1 < CNS:
            if sub >= 1:
                pltpu.make_async_copy(
                    other[0],
                    y_hbm.at[pl.ds(tok0 + (sub - 1) * CSUB, CSUB), :],
                    other[5]).wait()
            fire(sub + 1, 1 - s)

        @plsc.parallel_loop(0, CSUB * (IN_F // 16), unroll=8)
        def _add(i):
            r = i // (IN_F // 16)
            sl = pl.ds((i % (IN_F // 16)) * 16, 16)
            bufs[0][r, sl] = (bufs[0][r, sl] + bufs[1][r, sl]
                              + bufs[2][r, sl] + bufs[3][r, sl])

        pltpu.async_copy(bufs[0],
                         y_hbm.at[pl.ds(tok0 + sub * CSUB, CSUB), :],
                         bufs[5])
    for sub in (CNS - 2, CNS - 1):
        bufs = sets[sub & 1]
        pltpu.make_async_copy(bufs[0],
                              y_hbm.at[pl.ds(tok0 + sub * CSUB, CSUB), :],
                              bufs[5]).wait()


def _combine(yg, dest):
    mesh = plsc.VectorSubcoreMesh(core_axis_name="c", subcore_axis_name="s")
    fn = functools.partial(
        pl.kernel,
        out_type=jax.ShapeDtypeStruct((TOK, IN_F), jnp.float32),
        mesh=mesh,
        scratch_types=(
            [pltpu.VMEM((TOPK * CNS, CSUB), jnp.int32)]
            + [pltpu.VMEM((CSUB, IN_F), jnp.float32) for _ in range(8)]
            + [pltpu.SemaphoreType.DMA for _ in range(4)]
        ),
    )(_combine_body)
    return fn(yg, dest)


def kernel(x, Wr, br, W1, b1, W2, b2):
    x2 = x.reshape(TOK, IN_F)
    br2 = br.reshape(1, NBLK)
    b1b = b1.reshape(NBLK, 1, BLK)
    b22 = b2.reshape(1, IN_F)

    dest, bid = _router(x2, Wr, br2)
    xg = _dispatch(x2, dest)
    yg = _grouped_ffn(bid, xg, W1, b1b, W2, b22)
    y = _combine(yg, dest)
    return y.reshape(x.shape)
